# Initial kernel scaffold; baseline (speedup 1.0000x reference)
import jax
import jax.numpy as jnp
from jax.experimental import pallas as pl


def kernel(edge_index, edge_type, init_embed, init_rel):
    raise NotImplementedError("write your pallas kernel here")


# speedup vs baseline: 7.1572x; 7.1572x over previous
import dataclasses
import functools

import jax
import jax.numpy as jnp
from jax import lax
from jax.experimental import pallas as pl
from jax.experimental.pallas import tpu as pltpu
from jax.experimental.pallas import tpu_sc as plsc

NUM_ENT = 10000
NUM_REL = 200
DIM = 128
N_EDGES = 320000

NSUB = 16
E = 128
GRP = 8
CHUNKS = 160
PAD_EDGES = NSUB * CHUNKS * E
ACC_ROWS = 10240
ROWS_PER_SUB = ACC_ROWS // NSUB
HBLK = ACC_ROWS // DIM
TBL_ROWS = NUM_ENT + 2 * NUM_REL


def _sc_kernel(table, gidx, ridx, out,
               idxg, idxr, gbuf, hist, tslab, dslab, acc, stage):
    c = lax.axis_index("c")
    s = lax.axis_index("s")

    @pl.loop(0, E)
    def _(i):
        for g in range(DIM // 16):
            gbuf[i, pl.ds(16 * g, 16)] = jnp.zeros((16,), jnp.float32)

    @pl.loop(0, HBLK)
    def _(b):
        for g in range(DIM // 16):
            hist[b, pl.ds(16 * g, 16)] = jnp.zeros((16,), jnp.float32)

    @pl.loop(0, ROWS_PER_SUB // E)
    def _(z):
        base = s * ROWS_PER_SUB + z * E
        pltpu.sync_copy(gbuf, acc.at[pl.ds(base, E)])

    plsc.subcore_barrier()

    one16 = jnp.ones((16,), jnp.float32)

    @pl.loop(0, CHUNKS // GRP)
    def _(g):
        pltpu.sync_copy(gidx.at[c, s, pl.ds(g * GRP, GRP)], idxg)
        pltpu.sync_copy(ridx.at[s, pl.ds(g * GRP, GRP)], idxr)
        for j in range(GRP):
            pltpu.sync_copy(table.at[idxg.at[j]], gbuf)
            pltpu.sync_copy(gbuf, acc.at[idxr.at[j]], add=True)
            for q in range(E // 16):
                r16 = idxr[j, pl.ds(16 * q, 16)]
                plsc.addupdate_scatter(
                    hist,
                    [lax.shift_right_logical(r16, 7),
                     lax.bitwise_and(r16, 127)],
                    one16)

    @pl.loop(0, HBLK)
    def _(b):
        pltpu.sync_copy(hist.at[pl.ds(b, 1)], stage.at[b, pl.ds(s, 1)])

    plsc.subcore_barrier()

    @pl.loop(0, ROWS_PER_SUB // E)
    def _(z):
        base = s * ROWS_PER_SUB + z * E
        blk = s * (ROWS_PER_SUB // E) + z
        pltpu.sync_copy(acc.at[pl.ds(base, E)], gbuf)
        pltpu.sync_copy(stage.at[blk], tslab)

        for g in range(DIM // 16):
            d = tslab[0, pl.ds(16 * g, 16)]
            for t in range(1, NSUB):
                d = d + tslab[t, pl.ds(16 * g, 16)]
            dslab[0, pl.ds(16 * g, 16)] = 1.0 / jnp.where(d == 0.0, 1.0, d)

        @pl.loop(0, E)
        def _(i):
            r = plsc.load_gather(
                dslab, [jnp.zeros((16,), jnp.int32),
                        jnp.full((16,), i, jnp.int32)])
            for g in range(DIM // 16):
                gbuf[i, pl.ds(16 * g, 16)] = gbuf[i, pl.ds(16 * g, 16)] * r

        pltpu.sync_copy(gbuf, out.at[c, pl.ds(base, E)])


@jax.jit
def _run(table, gidx, ridx):
    mesh = plsc.VectorSubcoreMesh(core_axis_name="c", subcore_axis_name="s")
    cp = pltpu.CompilerParams()
    if "needs_layout_passes" in pltpu.CompilerParams.__dataclass_fields__:
        cp = dataclasses.replace(cp, needs_layout_passes=False)
    f = functools.partial(
        pl.kernel,
        out_type=jax.ShapeDtypeStruct((2, ACC_ROWS, DIM), jnp.float32),
        mesh=mesh,
        compiler_params=cp,
        scratch_types=[
            pltpu.VMEM((GRP, E), jnp.int32),
            pltpu.VMEM((GRP, E), jnp.int32),
            pltpu.VMEM((E, DIM), jnp.float32),
            pltpu.VMEM((HBLK, DIM), jnp.float32),
            pltpu.VMEM((NSUB, DIM), jnp.float32),
            pltpu.VMEM((1, DIM), jnp.float32),
            pltpu.VMEM_SHARED((ACC_ROWS, DIM), jnp.float32),
            pltpu.VMEM_SHARED((HBLK, NSUB, DIM), jnp.float32),
        ],
    )(_sc_kernel)
    return f(table, gidx, ridx)


def kernel(edge_index, edge_type, init_embed, init_rel):
    row = edge_index[0]
    col = edge_index[1]
    table = jnp.concatenate([init_embed, init_rel], axis=0)
    pad = PAD_EDGES - N_EDGES
    spread = jnp.arange(pad, dtype=jnp.int32)
    gidx_ent = jnp.concatenate([col, spread % TBL_ROWS])
    gidx_rel = jnp.concatenate([edge_type + NUM_ENT, spread % TBL_ROWS])
    gidx = jnp.stack([gidx_ent, gidx_rel]).reshape(2, NSUB, CHUNKS, E)
    ridx = jnp.concatenate(
        [row, NUM_ENT + spread % (ACC_ROWS - NUM_ENT)]).reshape(
            NSUB, CHUNKS, E)
    out = _run(table, gidx, ridx)
    return out[0, :NUM_ENT], out[1, :NUM_ENT]


# speedup vs baseline: 9.1496x; 1.2784x over previous
import dataclasses
import functools

import jax
import jax.numpy as jnp
from jax import lax
from jax.experimental import pallas as pl
from jax.experimental.pallas import tpu as pltpu
from jax.experimental.pallas import tpu_sc as plsc

NUM_ENT = 10000
NUM_REL = 200
DIM = 128
N_EDGES = 320000

NSUB = 16
E = 128
GRP = 8
CHUNKS = 160
PAD_EDGES = NSUB * CHUNKS * E
ACC_ROWS = 10240
ROWS_PER_SUB = ACC_ROWS // NSUB
HBLK = ACC_ROWS // DIM
TBL_ROWS = NUM_ENT + 2 * NUM_REL


def _sc_kernel(table, gidx, ridx, out,
               idxg, idxr, gbuf, hist, hidx, dslab, gsem, ssem, acc, deg):
    c = lax.axis_index("c")
    s = lax.axis_index("s")

    @pl.loop(0, E)
    def _(i):
        for g in range(DIM // 16):
            gbuf[0, i, pl.ds(16 * g, 16)] = jnp.zeros((16,), jnp.float32)

    @pl.loop(0, HBLK)
    def _(b):
        for g in range(DIM // 16):
            hist[b, pl.ds(16 * g, 16)] = jnp.zeros((16,), jnp.float32)

    for k in range(HBLK // 16):
        hidx[0, pl.ds(16 * k, 16)] = lax.iota(jnp.int32, 16) + 16 * k

    @pl.loop(0, ROWS_PER_SUB // E)
    def _(z):
        base = s * ROWS_PER_SUB + z * E
        pltpu.sync_copy(gbuf.at[0], acc.at[pl.ds(base, E)])

    pltpu.sync_copy(gbuf.at[0, pl.ds(0, HBLK // NSUB)],
                    deg.at[pl.ds(s * (HBLK // NSUB), HBLK // NSUB)])

    plsc.subcore_barrier()

    one16 = jnp.ones((16,), jnp.float32)

    @pl.loop(0, CHUNKS // GRP)
    def _(g):
        pltpu.sync_copy(gidx.at[c, s, pl.ds(g * GRP, GRP)], idxg)
        pltpu.sync_copy(ridx.at[s, pl.ds(g * GRP, GRP)], idxr)
        scat = [None, None]
        for j in range(GRP):
            b = j % 2
            if scat[b] is not None:
                scat[b].wait()
            gath = pltpu.async_copy(table.at[idxg.at[j]], gbuf.at[b], gsem)
            for q in range(E // 16):
                r16 = idxr[j, pl.ds(16 * q, 16)]
                plsc.addupdate_scatter(
                    hist,
                    [lax.shift_right_logical(r16, 7),
                     lax.bitwise_and(r16, 127)],
                    one16)
            gath.wait()
            scat[b] = pltpu.async_copy(gbuf.at[b], acc.at[idxr.at[j]], ssem,
                                       add=True)
        scat[0].wait()
        scat[1].wait()

    pltpu.sync_copy(hist, deg.at[hidx.at[0]], add=True)

    plsc.subcore_barrier()

    @pl.loop(0, ROWS_PER_SUB // E)
    def _(z):
        base = s * ROWS_PER_SUB + z * E
        blk = s * (ROWS_PER_SUB // E) + z
        pltpu.sync_copy(acc.at[pl.ds(base, E)], gbuf.at[0])
        pltpu.sync_copy(deg.at[pl.ds(blk, 1)], dslab)

        for g in range(DIM // 16):
            d = dslab[0, pl.ds(16 * g, 16)]
            dslab[0, pl.ds(16 * g, 16)] = 1.0 / jnp.where(d == 0.0, 1.0, d)

        @pl.loop(0, E)
        def _(i):
            r = plsc.load_gather(
                dslab, [jnp.zeros((16,), jnp.int32),
                        jnp.full((16,), i, jnp.int32)])
            for g in range(DIM // 16):
                gbuf[0, i, pl.ds(16 * g, 16)] = (
                    gbuf[0, i, pl.ds(16 * g, 16)] * r)

        pltpu.sync_copy(gbuf.at[0], out.at[c, pl.ds(base, E)])


@jax.jit
def _run(table, gidx, ridx):
    mesh = plsc.VectorSubcoreMesh(core_axis_name="c", subcore_axis_name="s")
    cp = pltpu.CompilerParams()
    if "needs_layout_passes" in pltpu.CompilerParams.__dataclass_fields__:
        cp = dataclasses.replace(cp, needs_layout_passes=False)
    f = functools.partial(
        pl.kernel,
        out_type=jax.ShapeDtypeStruct((2, ACC_ROWS, DIM), jnp.float32),
        mesh=mesh,
        compiler_params=cp,
        scratch_types=[
            pltpu.VMEM((GRP, E), jnp.int32),
            pltpu.VMEM((GRP, E), jnp.int32),
            pltpu.VMEM((2, E, DIM), jnp.float32),
            pltpu.VMEM((HBLK, DIM), jnp.float32),
            pltpu.VMEM((1, HBLK), jnp.int32),
            pltpu.VMEM((1, DIM), jnp.float32),
            pltpu.SemaphoreType.DMA,
            pltpu.SemaphoreType.DMA,
            pltpu.VMEM_SHARED((ACC_ROWS, DIM), jnp.float32),
            pltpu.VMEM_SHARED((HBLK, DIM), jnp.float32),
        ],
    )(_sc_kernel)
    return f(table, gidx, ridx)


def kernel(edge_index, edge_type, init_embed, init_rel):
    row = edge_index[0]
    col = edge_index[1]
    table = jnp.concatenate([init_embed, init_rel], axis=0)
    pad = PAD_EDGES - N_EDGES
    spread = jnp.arange(pad, dtype=jnp.int32)
    gidx_ent = jnp.concatenate([col, spread % TBL_ROWS])
    gidx_rel = jnp.concatenate([edge_type + NUM_ENT, spread % TBL_ROWS])
    gidx = jnp.stack([gidx_ent, gidx_rel]).reshape(2, NSUB, CHUNKS, E)
    ridx = jnp.concatenate(
        [row, NUM_ENT + spread % (ACC_ROWS - NUM_ENT)]).reshape(
            NSUB, CHUNKS, E)
    out = _run(table, gidx, ridx)
    return out[0, :NUM_ENT], out[1, :NUM_ENT]


# speedup vs baseline: 9.6758x; 1.0575x over previous
import dataclasses
import functools

import jax
import jax.numpy as jnp
from jax import lax
from jax.experimental import pallas as pl
from jax.experimental.pallas import tpu as pltpu
from jax.experimental.pallas import tpu_sc as plsc

NUM_ENT = 10000
NUM_REL = 200
DIM = 128
N_EDGES = 320000

NSUB = 16
E = 128
GRP = 8
CHUNKS = 160
PAD_EDGES = NSUB * CHUNKS * E
ACC_ROWS = 10240
ROWS_PER_SUB = ACC_ROWS // NSUB
HBLK = ACC_ROWS // DIM
TBL_ROWS = NUM_ENT + 2 * NUM_REL


def _sc_kernel(table, gidx, ridx, out,
               idxg, idxr, gbuf, hist, hidx, dslab, gsem, ssem, isem,
               acc, deg):
    c = lax.axis_index("c")
    s = lax.axis_index("s")

    @pl.loop(0, E)
    def _(i):
        for g in range(DIM // 16):
            gbuf[0, i, pl.ds(16 * g, 16)] = jnp.zeros((16,), jnp.float32)

    @pl.loop(0, HBLK)
    def _(b):
        for g in range(DIM // 16):
            hist[b, pl.ds(16 * g, 16)] = jnp.zeros((16,), jnp.float32)

    for k in range(HBLK // 16):
        hidx[0, pl.ds(16 * k, 16)] = lax.iota(jnp.int32, 16) + 16 * k

    @pl.loop(0, ROWS_PER_SUB // E)
    def _(z):
        base = s * ROWS_PER_SUB + z * E
        pltpu.sync_copy(gbuf.at[0], acc.at[pl.ds(base, E)])

    pltpu.sync_copy(gbuf.at[0, pl.ds(0, HBLK // NSUB)],
                    deg.at[pl.ds(s * (HBLK // NSUB), HBLK // NSUB)])

    plsc.subcore_barrier()

    one16 = jnp.ones((16,), jnp.float32)

    pltpu.sync_copy(gidx.at[c, s, pl.ds(0, GRP)], idxg.at[0])
    pltpu.sync_copy(ridx.at[s, pl.ds(0, GRP)], idxr.at[0])

    @pl.loop(0, CHUNKS // GRP)
    def _(g):
        ib = lax.rem(g, 2)
        nxt = lax.rem(g + 1, 2)
        last = g == CHUNKS // GRP - 1
        gnext = jnp.where(last, g, g + 1)
        ldg = pltpu.async_copy(gidx.at[c, s, pl.ds(gnext * GRP, GRP)],
                               idxg.at[nxt], isem)
        ldr = pltpu.async_copy(ridx.at[s, pl.ds(gnext * GRP, GRP)],
                               idxr.at[nxt], isem)
        scat = [None, None]
        for j in range(GRP):
            b = j % 2
            if scat[b] is not None:
                scat[b].wait()
            gath = pltpu.async_copy(table.at[idxg.at[ib, j]], gbuf.at[b],
                                    gsem)
            for q in range(E // 16):
                r16 = idxr[ib, j, pl.ds(16 * q, 16)]
                plsc.addupdate_scatter(
                    hist,
                    [lax.shift_right_logical(r16, 7),
                     lax.bitwise_and(r16, 127)],
                    one16)
            gath.wait()
            scat[b] = pltpu.async_copy(gbuf.at[b], acc.at[idxr.at[ib, j]],
                                       ssem, add=True)
        ldg.wait()
        ldr.wait()
        scat[0].wait()
        scat[1].wait()

    pltpu.sync_copy(hist, deg.at[hidx.at[0]], add=True)

    plsc.subcore_barrier()

    @pl.loop(0, ROWS_PER_SUB // E)
    def _(z):
        base = s * ROWS_PER_SUB + z * E
        blk = s * (ROWS_PER_SUB // E) + z
        pltpu.sync_copy(acc.at[pl.ds(base, E)], gbuf.at[0])
        pltpu.sync_copy(deg.at[pl.ds(blk, 1)], dslab)

        for g in range(DIM // 16):
            d = dslab[0, pl.ds(16 * g, 16)]
            dslab[0, pl.ds(16 * g, 16)] = 1.0 / jnp.where(d == 0.0, 1.0, d)

        @pl.loop(0, E)
        def _(i):
            r = plsc.load_gather(
                dslab, [jnp.zeros((16,), jnp.int32),
                        jnp.full((16,), i, jnp.int32)])
            for g in range(DIM // 16):
                gbuf[0, i, pl.ds(16 * g, 16)] = (
                    gbuf[0, i, pl.ds(16 * g, 16)] * r)

        pltpu.sync_copy(gbuf.at[0], out.at[c, pl.ds(base, E)])


@jax.jit
def _run(table, gidx, ridx):
    mesh = plsc.VectorSubcoreMesh(core_axis_name="c", subcore_axis_name="s")
    cp = pltpu.CompilerParams()
    if "needs_layout_passes" in pltpu.CompilerParams.__dataclass_fields__:
        cp = dataclasses.replace(cp, needs_layout_passes=False)
    f = functools.partial(
        pl.kernel,
        out_type=jax.ShapeDtypeStruct((2, ACC_ROWS, DIM), jnp.float32),
        mesh=mesh,
        compiler_params=cp,
        scratch_types=[
            pltpu.VMEM((2, GRP, E), jnp.int32),
            pltpu.VMEM((2, GRP, E), jnp.int32),
            pltpu.VMEM((2, E, DIM), jnp.float32),
            pltpu.VMEM((HBLK, DIM), jnp.float32),
            pltpu.VMEM((1, HBLK), jnp.int32),
            pltpu.VMEM((1, DIM), jnp.float32),
            pltpu.SemaphoreType.DMA,
            pltpu.SemaphoreType.DMA,
            pltpu.SemaphoreType.DMA,
            pltpu.VMEM_SHARED((ACC_ROWS, DIM), jnp.float32),
            pltpu.VMEM_SHARED((HBLK, DIM), jnp.float32),
        ],
    )(_sc_kernel)
    return f(table, gidx, ridx)


def kernel(edge_index, edge_type, init_embed, init_rel):
    row = edge_index[0]
    col = edge_index[1]
    table = jnp.concatenate([init_embed, init_rel], axis=0)
    pad = PAD_EDGES - N_EDGES
    spread = jnp.arange(pad, dtype=jnp.int32)
    gidx_ent = jnp.concatenate([col, spread % TBL_ROWS])
    gidx_rel = jnp.concatenate([edge_type + NUM_ENT, spread % TBL_ROWS])
    gidx = jnp.stack([gidx_ent, gidx_rel]).reshape(2, NSUB, CHUNKS, E)
    ridx = jnp.concatenate(
        [row, NUM_ENT + spread % (ACC_ROWS - NUM_ENT)]).reshape(
            NSUB, CHUNKS, E)
    out = _run(table, gidx, ridx)
    return out[0, :NUM_ENT], out[1, :NUM_ENT]
